# baseline (speedup 1.0000x reference)
import functools

import jax
import jax.numpy as jnp
from jax import lax
from jax.experimental import pallas as pl
from jax.experimental.pallas import tpu as pltpu
from jax.experimental.pallas import tpu_sc as plsc

NC = 2
NS = 16
NW = NC * NS
L = 16

VOCAB = 1000000
DIM = 64
BATCH = 16384
NUM_CTX = 5

CHUNK = 128
ROWS_PER_W = BATCH // NW
N_CHUNKS = ROWS_PER_W // CHUNK


GRP = 16
N_GRP = CHUNK // GRP
PAIRS = GRP * NUM_CTX


def _sc_kernel(sg_table, ctx_table, target, context_t, out,
               tgt_idx, ctx_idx, tgt_rows, ctx_rows, prods, out_v, sem):
  wid = lax.axis_index("s") * NC + lax.axis_index("c")
  iota = lax.iota(jnp.int32, L)
  gather_base = iota * L

  for c in range(N_CHUNKS):
    base = wid * ROWS_PER_W + c * CHUNK

    pltpu.sync_copy(target.at[pl.ds(base, CHUNK)], tgt_idx)
    for j in range(NUM_CTX):
      pltpu.sync_copy(context_t.at[pl.ds(j * BATCH + base, CHUNK)],
                      ctx_idx.at[pl.ds(j * CHUNK, CHUNK)])

    copies = [pltpu.async_copy(sg_table.at[tgt_idx], tgt_rows, sem)]
    for j in range(NUM_CTX):
      copies.append(
          pltpu.async_copy(ctx_table.at[ctx_idx.at[pl.ds(j * CHUNK, CHUNK)]],
                           ctx_rows.at[pl.ds(j * CHUNK, CHUNK)], sem))
    for cp in copies:
      cp.wait()

    def body(g, carry):
      b0 = g * GRP
      for bi in range(GRP):
        b = b0 + bi
        tb = [tgt_rows[b, pl.ds(k * L, L)] for k in range(DIM // L)]
        for j in range(NUM_CTX):
          r = j * CHUNK + b
          acc = ctx_rows[r, pl.ds(0, L)] * tb[0]
          for k in range(1, DIM // L):
            acc = acc + ctx_rows[r, pl.ds(k * L, L)] * tb[k]
          prods[pl.ds((bi * NUM_CTX + j) * L, L)] = acc
      for o in range(PAIRS // L):
        sums = plsc.load_gather(prods, [gather_base + o * (L * L)])
        for k in range(1, L):
          sums = sums + plsc.load_gather(
              prods, [gather_base + (o * (L * L) + k)])
        out_v[pl.ds(b0 * NUM_CTX + o * L, L)] = sums
      return carry

    lax.fori_loop(0, N_GRP, body, 0)

    pltpu.sync_copy(out_v, out.at[pl.ds(base * NUM_CTX, CHUNK * NUM_CTX)])


@jax.jit
def _run(target, context_t, sg_table, ctx_table):
  mesh = plsc.VectorSubcoreMesh(core_axis_name="c", subcore_axis_name="s")
  return pl.kernel(
      _sc_kernel,
      out_type=jax.ShapeDtypeStruct((BATCH * NUM_CTX,), jnp.float32),
      mesh=mesh,
      compiler_params=pltpu.CompilerParams(
          needs_layout_passes=False, use_tc_tiling_on_sc=False),
      scratch_types=[
          pltpu.VMEM((CHUNK,), jnp.int32),
          pltpu.VMEM((NUM_CTX * CHUNK,), jnp.int32),
          pltpu.VMEM((CHUNK, DIM), jnp.float32),
          pltpu.VMEM((NUM_CTX * CHUNK, DIM), jnp.float32),
          pltpu.VMEM((PAIRS * L,), jnp.float32),
          pltpu.VMEM((CHUNK * NUM_CTX,), jnp.float32),
          pltpu.SemaphoreType.DMA,
      ],
  )(sg_table, ctx_table, target, context_t)


def kernel(target, context, sg_table, ctx_table):
  context_t = jnp.transpose(context.astype(jnp.int32), (1, 0)).reshape(-1)
  out_flat = _run(target.astype(jnp.int32), context_t, sg_table, ctx_table)
  return out_flat.reshape(BATCH, NUM_CTX)
